# baseline (speedup 1.0000x reference)
import functools

import jax
import jax.numpy as jnp
from jax import lax
from jax.experimental import pallas as pl
from jax.experimental.pallas import tpu as pltpu
from jax.experimental.pallas import tpu_sc as plsc

_INT_MIN = -2147483648
_FLIP = 0x7FFFFFFF


def _encode_body(x_ref, w_ref, b_ref, out_ref):
    out_ref[...] = (
        jnp.dot(x_ref[...], w_ref[0], preferred_element_type=jnp.float32)
        + b_ref[...]
    )


def _monotone_key(v):
    key = lax.bitcast_convert_type(v, jnp.int32)
    return jnp.where(key < 0, key ^ jnp.int32(_FLIP), key)


_HU = 8


def _sc_select_body(pre_hbm, kk_hbm, out_hbm, row_v, cidx_v, hist16_v, hist_v,
                    suf_v, kk_v, out_v, *, n_row):
    nv = n_row // (16 * _HU)
    wid = lax.axis_index("s") * 2 + lax.axis_index("c")

    pltpu.sync_copy(kk_hbm, kk_v)
    pltpu.sync_copy(pre_hbm.at[pl.ds(wid * n_row, n_row)], row_v)
    kk = kk_v[...]
    int_min = jnp.int32(_INT_MIN)
    ones = jnp.ones((16,), jnp.int32)
    zeros16 = jnp.zeros((16,), jnp.int32)
    lane = lax.broadcasted_iota(jnp.int32, (16,), 0)
    lane_base = lane * 256

    def zero_hists():
        for j in range(256):
            hist16_v[pl.ds(j * 16, 16)] = zeros16

    def merge_hists():
        for j in range(16):
            acc = hist16_v[pl.ds(j * 16, 16)]
            for l in range(1, 16):
                acc = acc + hist16_v[pl.ds(l * 256 + j * 16, 16)]
            hist_v[pl.ds(j * 16, 16)] = acc

    def pick_digit(kk_rem):
        carry = zeros16
        nsat = zeros16
        for j in range(15, -1, -1):
            h = hist_v[pl.ds(j * 16, 16)]
            suf = lax.rev(plsc.cumsum(lax.rev(h, (0,))), (0,)) + carry
            suf_v[pl.ds(j * 16, 16)] = suf
            carry = jnp.broadcast_to(jnp.max(suf), (16,))
            nsat = nsat + plsc.all_reduce_population_count(suf >= kk_rem)
        dstar = nsat - 1
        sd = plsc.load_gather(suf_v, [dstar])
        hd = plsc.load_gather(hist_v, [dstar])
        return dstar, kk_rem - (sd - hd), hd

    zero_hists()

    def p1(i, c):
        ws = [row_v[pl.ds((i * _HU + u) * 16, 16)] for u in range(_HU)]
        ubs = [_monotone_key(w) ^ int_min for w in ws]
        digits = [lax.shift_right_logical(ub, 24) + lane_base for ub in ubs]
        for u in range(_HU):
            plsc.addupdate_scatter(hist16_v, [digits[u]], ones)
        return c

    lax.fori_loop(0, nv, p1, jnp.int32(0))
    merge_hists()
    dstar, kk_rem, hd = pick_digit(kk)
    prefix = lax.shift_left(dstar, 24)

    for sh in (16, 8, 0):
        zero_hists()
        pref_hi = lax.shift_right_logical(prefix, sh + 8)

        def pk(i, c, sh=sh, pref_hi=pref_hi):
            ws = [row_v[pl.ds((i * _HU + u) * 16, 16)] for u in range(_HU)]
            ubs = [_monotone_key(w) ^ int_min for w in ws]
            msks = [lax.shift_right_logical(ub, sh + 8) == pref_hi
                    for ub in ubs]
            digits = [jnp.bitwise_and(lax.shift_right_logical(ub, sh),
                                      jnp.int32(255)) + lane_base
                      for ub in ubs]
            for u in range(_HU):
                plsc.addupdate_scatter(hist16_v, [digits[u]], ones,
                                       mask=msks[u])
            return c

        lax.fori_loop(0, nv, pk, jnp.int32(0))
        merge_hists()
        dstar, kk_rem, hd = pick_digit(kk_rem)
        prefix = prefix | lax.shift_left(dstar, sh)

    thresh = prefix ^ int_min
    need = kk_rem
    n_eq = hd

    need_s = jnp.max(need)
    neq_s = jnp.max(n_eq)

    def tie_break():
        def cbody(i, off):
            w = row_v[pl.ds(i * 16, 16)]
            ub = _monotone_key(w) ^ int_min
            eq = ub == prefix
            plsc.store_compressed(cidx_v.at[pl.ds(off, 16)], i * 16 + lane,
                                  mask=eq)
            return off + jnp.sum(jnp.where(eq, 1, 0))

        lax.fori_loop(0, n_row // 16, cbody, jnp.int32(0))
        safe = jnp.broadcast_to(jnp.maximum(need_s - 1, 0), (16,))
        got = plsc.load_gather(cidx_v, [safe])
        return jnp.where(need_s > 0, jnp.max(got) + 1, jnp.int32(0))

    mcut = lax.cond(need_s < neq_s, tie_break,
                    lambda: jnp.int32(n_row))

    out_v[...] = jnp.where(lane == 0, thresh,
                           jnp.where(lane == 1, jnp.broadcast_to(mcut, (16,)),
                                     0))
    pltpu.sync_copy(out_v, out_hbm.at[pl.ds(wid * 16, 16)])


def _decode_body(th_ref, pre_ref, w_ref, b_ref, x_ref, z_ref, xhat_ref,
                 loss_ref, *, n_tiles, d_sae, inv_btl):
    tl = pl.program_id(0)
    thresh = th_ref[:, 0:1]
    mcut = th_ref[:, 1:2]
    v = pre_ref[...]
    skey = _monotone_key(v)
    idx = lax.broadcasted_iota(jnp.int32, v.shape, 1) + tl * d_sae
    mask = (skey > thresh) | ((skey == thresh) & (idx < mcut))
    z = jnp.where(mask, jnp.maximum(v, 0.0), 0.0)
    z_ref[...] = z
    xh = (
        jnp.dot(z, w_ref[0], preferred_element_type=jnp.float32)
        + b_ref[0]
    )
    xhat_ref[...] = xh
    d = xh - x_ref[...]
    s = jnp.sum(d * d)

    @pl.when(tl == 0)
    def _init():
        loss_ref[0, 0] = 0.0

    loss_ref[0, 0] += s

    @pl.when(tl == n_tiles - 1)
    def _final():
        loss_ref[0, 0] = loss_ref[0, 0] * inv_btl


def kernel(x, W_enc, W_dec, b_enc, b_dec, k):
    B, T, L, d_in = x.shape
    d_sae = W_enc.shape[-1]
    TL = T * L
    N = TL * d_sae
    K_STATIC = 512

    x2 = x.reshape(B, TL * d_in)
    we = W_enc.reshape(TL, d_in, d_sae)
    wd = W_dec.reshape(TL, d_sae, d_in)
    be = b_enc.reshape(1, d_sae)
    bd = b_dec.reshape(TL, 1, d_in)
    kk = jnp.clip(jnp.asarray(k, jnp.int32), 0, K_STATIC)
    kkvec = jnp.full((16,), kk, jnp.int32)

    pre = pl.pallas_call(
        _encode_body,
        grid=(TL,),
        in_specs=[
            pl.BlockSpec((B, d_in), lambda i: (0, i)),
            pl.BlockSpec((1, d_in, d_sae), lambda i: (i, 0, 0)),
            pl.BlockSpec((1, d_sae), lambda i: (0, 0)),
        ],
        out_specs=pl.BlockSpec((B, d_sae), lambda i: (0, i)),
        out_shape=jax.ShapeDtypeStruct((B, N), jnp.float32),
    )(x2, we, be)

    sc_select = pl.kernel(
        functools.partial(_sc_select_body, n_row=N),
        out_type=jax.ShapeDtypeStruct((B * 16,), jnp.int32),
        mesh=plsc.VectorSubcoreMesh(core_axis_name="c", subcore_axis_name="s"),
        compiler_params=pltpu.CompilerParams(needs_layout_passes=False),
        scratch_types=[
            pltpu.VMEM((N,), jnp.float32),
            pltpu.VMEM((N + 16,), jnp.int32),
            pltpu.VMEM((4096,), jnp.int32),
            pltpu.VMEM((256,), jnp.int32),
            pltpu.VMEM((256,), jnp.int32),
            pltpu.VMEM((16,), jnp.int32),
            pltpu.VMEM((16,), jnp.int32),
        ],
    )
    th = sc_select(pre.reshape(B * N), kkvec).reshape(B, 16)

    z2, xhat2, loss = pl.pallas_call(
        functools.partial(_decode_body, n_tiles=TL, d_sae=d_sae,
                          inv_btl=1.0 / float(B * TL)),
        grid=(TL,),
        in_specs=[
            pl.BlockSpec((B, 16), lambda i: (0, 0)),
            pl.BlockSpec((B, d_sae), lambda i: (0, i)),
            pl.BlockSpec((1, d_sae, d_in), lambda i: (i, 0, 0)),
            pl.BlockSpec((1, 1, d_in), lambda i: (i, 0, 0)),
            pl.BlockSpec((B, d_in), lambda i: (0, i)),
        ],
        out_specs=[
            pl.BlockSpec((B, d_sae), lambda i: (0, i)),
            pl.BlockSpec((B, d_in), lambda i: (0, i)),
            pl.BlockSpec(memory_space=pltpu.SMEM),
        ],
        out_shape=[
            jax.ShapeDtypeStruct((B, N), jnp.float32),
            jax.ShapeDtypeStruct((B, TL * d_in), jnp.float32),
            jax.ShapeDtypeStruct((1, 1), jnp.float32),
        ],
    )(th, pre, wd, bd, x2)

    x_hat = xhat2.reshape(B, T, L, d_in)
    z = z2.reshape(B, T, L, d_sae)
    return (loss.reshape(()), x_hat, z)


# speedup vs baseline: 1.0389x; 1.0389x over previous
import functools

import jax
import jax.numpy as jnp
from jax import lax
from jax.experimental import pallas as pl
from jax.experimental.pallas import tpu as pltpu
from jax.experimental.pallas import tpu_sc as plsc

_INT_MIN = -2147483648
_FLIP = 0x7FFFFFFF


def _encode_body(x_ref, w_ref, b_ref, out_ref):
    out_ref[...] = (
        jnp.dot(x_ref[...], w_ref[0], preferred_element_type=jnp.float32)
        + b_ref[...]
    )


def _monotone_key(v):
    key = lax.bitcast_convert_type(v, jnp.int32)
    return jnp.where(key < 0, key ^ jnp.int32(_FLIP), key)


_HU = 8


_NCHUNK = 4


def _sc_select_body(pre_hbm, kk_hbm, out_hbm, row_v, cidx_v, hist_v, suf_v,
                    kk_v, out_v, *rest, n_row):
    hists = rest[:_HU]
    sems = rest[_HU:]
    nv = n_row // (16 * _HU)
    wid = lax.axis_index("s") * 2 + lax.axis_index("c")

    chunk = n_row // _NCHUNK
    copies = [
        pltpu.async_copy(
            pre_hbm.at[pl.ds(wid * n_row + c * chunk, chunk)],
            row_v.at[pl.ds(c * chunk, chunk)],
            sems[c],
        )
        for c in range(_NCHUNK)
    ]
    pltpu.sync_copy(kk_hbm, kk_v)
    kk = kk_v[...]
    int_min = jnp.int32(_INT_MIN)
    ones = jnp.ones((16,), jnp.int32)
    zeros16 = jnp.zeros((16,), jnp.int32)
    lane = lax.broadcasted_iota(jnp.int32, (16,), 0)

    def zero_hists():
        for h in hists:
            for j in range(16):
                h[pl.ds(j * 16, 16)] = zeros16

    def merge_hists():
        for j in range(16):
            acc = hists[0][pl.ds(j * 16, 16)]
            for u in range(1, _HU):
                acc = acc + hists[u][pl.ds(j * 16, 16)]
            hist_v[pl.ds(j * 16, 16)] = acc

    def pick_digit(kk_rem):
        carry = zeros16
        nsat = zeros16
        for j in range(15, -1, -1):
            h = hist_v[pl.ds(j * 16, 16)]
            suf = lax.rev(plsc.cumsum(lax.rev(h, (0,))), (0,)) + carry
            suf_v[pl.ds(j * 16, 16)] = suf
            carry = jnp.broadcast_to(jnp.max(suf), (16,))
            nsat = nsat + plsc.all_reduce_population_count(suf >= kk_rem)
        dstar = nsat - 1
        sd = plsc.load_gather(suf_v, [dstar])
        hd = plsc.load_gather(hist_v, [dstar])
        return dstar, kk_rem - (sd - hd), hd

    zero_hists()

    def p1(i, c):
        ws = [row_v[pl.ds((i * _HU + u) * 16, 16)] for u in range(_HU)]
        ubs = [_monotone_key(w) ^ int_min for w in ws]
        digits = [lax.shift_right_logical(ub, 24) for ub in ubs]
        for u in range(_HU):
            plsc.addupdate_scatter(hists[u], [digits[u]], ones)
        return c

    nvc = nv // _NCHUNK
    for c in range(_NCHUNK):
        copies[c].wait()
        lax.fori_loop(c * nvc, (c + 1) * nvc, p1, jnp.int32(0))
    merge_hists()
    dstar, kk_rem, hd = pick_digit(kk)
    prefix = lax.shift_left(dstar, 24)

    for sh in (16, 8, 0):
        zero_hists()
        pref_hi = lax.shift_right_logical(prefix, sh + 8)

        def pk(i, c, sh=sh, pref_hi=pref_hi):
            ws = [row_v[pl.ds((i * _HU + u) * 16, 16)] for u in range(_HU)]
            ubs = [_monotone_key(w) ^ int_min for w in ws]
            msks = [lax.shift_right_logical(ub, sh + 8) == pref_hi
                    for ub in ubs]
            digits = [jnp.bitwise_and(lax.shift_right_logical(ub, sh),
                                      jnp.int32(255)) for ub in ubs]
            for u in range(_HU):
                plsc.addupdate_scatter(hists[u], [digits[u]], ones,
                                       mask=msks[u])
            return c

        lax.fori_loop(0, nv, pk, jnp.int32(0))
        merge_hists()
        dstar, kk_rem, hd = pick_digit(kk_rem)
        prefix = prefix | lax.shift_left(dstar, sh)

    thresh = prefix ^ int_min
    need = kk_rem
    n_eq = hd

    need_s = jnp.max(need)
    neq_s = jnp.max(n_eq)

    def tie_break():
        def cbody(i, off):
            w = row_v[pl.ds(i * 16, 16)]
            ub = _monotone_key(w) ^ int_min
            eq = ub == prefix
            plsc.store_compressed(cidx_v.at[pl.ds(off, 16)], i * 16 + lane,
                                  mask=eq)
            return off + jnp.sum(jnp.where(eq, 1, 0))

        lax.fori_loop(0, n_row // 16, cbody, jnp.int32(0))
        safe = jnp.broadcast_to(jnp.maximum(need_s - 1, 0), (16,))
        got = plsc.load_gather(cidx_v, [safe])
        return jnp.where(need_s > 0, jnp.max(got) + 1, jnp.int32(0))

    mcut = lax.cond(need_s < neq_s, tie_break,
                    lambda: jnp.int32(n_row))

    out_v[...] = jnp.where(lane == 0, thresh,
                           jnp.where(lane == 1, jnp.broadcast_to(mcut, (16,)),
                                     0))
    pltpu.sync_copy(out_v, out_hbm.at[pl.ds(wid * 16, 16)])


def _decode_body(th_ref, pre_ref, w_ref, b_ref, x_ref, z_ref, xhat_ref,
                 loss_ref, *, n_tiles, d_sae, inv_btl):
    tl = pl.program_id(0)
    thresh = th_ref[:, 0:1]
    mcut = th_ref[:, 1:2]
    v = pre_ref[...]
    skey = _monotone_key(v)
    idx = lax.broadcasted_iota(jnp.int32, v.shape, 1) + tl * d_sae
    mask = (skey > thresh) | ((skey == thresh) & (idx < mcut))
    z = jnp.where(mask, jnp.maximum(v, 0.0), 0.0)
    z_ref[...] = z
    xh = (
        jnp.dot(z, w_ref[0], preferred_element_type=jnp.float32)
        + b_ref[0]
    )
    xhat_ref[...] = xh
    d = xh - x_ref[...]
    s = jnp.sum(d * d)

    @pl.when(tl == 0)
    def _init():
        loss_ref[0, 0] = 0.0

    loss_ref[0, 0] += s

    @pl.when(tl == n_tiles - 1)
    def _final():
        loss_ref[0, 0] = loss_ref[0, 0] * inv_btl


def kernel(x, W_enc, W_dec, b_enc, b_dec, k):
    B, T, L, d_in = x.shape
    d_sae = W_enc.shape[-1]
    TL = T * L
    N = TL * d_sae
    K_STATIC = 512

    x2 = x.reshape(B, TL * d_in)
    we = W_enc.reshape(TL, d_in, d_sae)
    wd = W_dec.reshape(TL, d_sae, d_in)
    be = b_enc.reshape(1, d_sae)
    bd = b_dec.reshape(TL, 1, d_in)
    kk = jnp.clip(jnp.asarray(k, jnp.int32), 0, K_STATIC)
    kkvec = jnp.full((16,), kk, jnp.int32)

    pre = pl.pallas_call(
        _encode_body,
        grid=(TL,),
        in_specs=[
            pl.BlockSpec((B, d_in), lambda i: (0, i)),
            pl.BlockSpec((1, d_in, d_sae), lambda i: (i, 0, 0)),
            pl.BlockSpec((1, d_sae), lambda i: (0, 0)),
        ],
        out_specs=pl.BlockSpec((B, d_sae), lambda i: (0, i)),
        out_shape=jax.ShapeDtypeStruct((B, N), jnp.float32),
    )(x2, we, be)

    sc_select = pl.kernel(
        functools.partial(_sc_select_body, n_row=N),
        out_type=jax.ShapeDtypeStruct((B * 16,), jnp.int32),
        mesh=plsc.VectorSubcoreMesh(core_axis_name="c", subcore_axis_name="s"),
        compiler_params=pltpu.CompilerParams(needs_layout_passes=False),
        scratch_types=[
            pltpu.VMEM((N,), jnp.float32),
            pltpu.VMEM((N + 16,), jnp.int32),
            pltpu.VMEM((256,), jnp.int32),
            pltpu.VMEM((256,), jnp.int32),
            pltpu.VMEM((16,), jnp.int32),
            pltpu.VMEM((16,), jnp.int32),
        ] + [pltpu.VMEM((256,), jnp.int32) for _ in range(_HU)]
          + [pltpu.SemaphoreType.DMA for _ in range(_NCHUNK)],
    )
    th = sc_select(pre.reshape(B * N), kkvec).reshape(B, 16)

    z2, xhat2, loss = pl.pallas_call(
        functools.partial(_decode_body, n_tiles=TL, d_sae=d_sae,
                          inv_btl=1.0 / float(B * TL)),
        grid=(TL,),
        in_specs=[
            pl.BlockSpec((B, 16), lambda i: (0, 0)),
            pl.BlockSpec((B, d_sae), lambda i: (0, i)),
            pl.BlockSpec((1, d_sae, d_in), lambda i: (i, 0, 0)),
            pl.BlockSpec((1, 1, d_in), lambda i: (i, 0, 0)),
            pl.BlockSpec((B, d_in), lambda i: (0, i)),
        ],
        out_specs=[
            pl.BlockSpec((B, d_sae), lambda i: (0, i)),
            pl.BlockSpec((B, d_in), lambda i: (0, i)),
            pl.BlockSpec(memory_space=pltpu.SMEM),
        ],
        out_shape=[
            jax.ShapeDtypeStruct((B, N), jnp.float32),
            jax.ShapeDtypeStruct((B, TL * d_in), jnp.float32),
            jax.ShapeDtypeStruct((1, 1), jnp.float32),
        ],
    )(th, pre, wd, bd, x2)

    x_hat = xhat2.reshape(B, T, L, d_in)
    z = z2.reshape(B, T, L, d_sae)
    return (loss.reshape(()), x_hat, z)
